# Initial kernel scaffold; baseline (speedup 1.0000x reference)
import functools

import jax
import jax.numpy as jnp
from jax import lax
from jax.experimental import pallas as pl
from jax.experimental.pallas import tpu as pltpu
from jax.experimental.pallas import tpu_sc as plsc

B = 256
N_IN = 16384
N_OUT = 16384
FAN = 16
L = 16

NC = 2
NS = 16
NW = NC * NS
B_PER_W = B // NW
NB = 4
NPASS = B_PER_W // NB
RB = 512
NBLK = N_OUT // RB
CHUNKS = RB // L


@functools.partial(
    pl.kernel,
    out_type=jax.ShapeDtypeStruct((B, N_OUT), jnp.float32),
    mesh=plsc.VectorSubcoreMesh(core_axis_name="c", subcore_axis_name="s"),
    scratch_types=[
        pltpu.VMEM((NB, N_IN), jnp.float32),
        pltpu.VMEM((FAN, RB), jnp.int32),
        pltpu.VMEM((FAN, RB), jnp.float32),
        pltpu.VMEM((RB,), jnp.float32),
        pltpu.VMEM((NB, RB), jnp.float32),
    ],
)
def _local_linear_sc(x_hbm, colsT_hbm, wT_hbm, bias_hbm, out_hbm,
                     x_v, c_v, w_v, b_v, o_v):
    wid = lax.axis_index("s") * NC + lax.axis_index("c")

    for p in range(NPASS):
        b0 = wid * B_PER_W + p * NB
        pltpu.sync_copy(x_hbm.at[pl.ds(b0, NB), :], x_v)

        def blk_body(j, _, b0=b0):
            c0 = j * RB
            pltpu.sync_copy(colsT_hbm.at[:, pl.ds(c0, RB)], c_v)
            pltpu.sync_copy(wT_hbm.at[:, pl.ds(c0, RB)], w_v)
            pltpu.sync_copy(bias_hbm.at[pl.ds(c0, RB)], b_v)

            def chunk_body(t, _):
                cc = t * L
                acc0 = b_v[pl.ds(cc, L)]
                accs = [acc0] * NB
                for k in range(FAN):
                    idx = c_v[k, pl.ds(cc, L)]
                    wv = w_v[k, pl.ds(cc, L)]
                    for bb in range(NB):
                        xv = plsc.load_gather(
                            x_v, [jnp.full((L,), bb, jnp.int32), idx])
                        accs[bb] = accs[bb] + xv * wv
                for bb in range(NB):
                    o_v[bb, pl.ds(cc, L)] = accs[bb]
                return 0

            lax.fori_loop(0, CHUNKS, chunk_body, 0)
            pltpu.sync_copy(o_v, out_hbm.at[pl.ds(b0, NB), pl.ds(c0, RB)])
            return 0

        lax.fori_loop(0, NBLK, blk_body, 0)


def kernel(x, rows, cols, weight, bias):
    del rows
    colsT = jnp.asarray(cols, jnp.int32).reshape(N_OUT, FAN).T
    wT = weight.reshape(N_OUT, FAN).T
    return _local_linear_sc(x, colsT, wT, bias)


# speedup vs baseline: 5.8859x; 5.8859x over previous
import functools

import jax
import jax.numpy as jnp
from jax import lax
from jax.experimental import pallas as pl
from jax.experimental.pallas import tpu as pltpu
from jax.experimental.pallas import tpu_sc as plsc

B = 256
N_IN = 16384
N_OUT = 16384
FAN = 16
L = 16

NC = 2
NS = 16
NW = NC * NS
B_PER_W = B // NW
NB = 4
NPASS = B_PER_W // NB
RB = 512
NBLK = N_OUT // RB
CHUNKS = RB // L


@functools.partial(
    pl.kernel,
    out_type=jax.ShapeDtypeStruct((B, N_OUT), jnp.float32),
    mesh=plsc.VectorSubcoreMesh(core_axis_name="c", subcore_axis_name="s"),
    compiler_params=pltpu.CompilerParams(needs_layout_passes=False),
    scratch_types=[
        pltpu.VMEM((NB * N_IN,), jnp.float32),
        pltpu.VMEM((FAN, RB), jnp.int32),
        pltpu.VMEM((FAN, RB), jnp.float32),
        pltpu.VMEM((RB,), jnp.float32),
        pltpu.VMEM((NB, RB), jnp.float32),
    ],
)
def _local_linear_sc(x_hbm, colsT_hbm, wT_hbm, bias_hbm, out_hbm,
                     x_v, c_v, w_v, b_v, o_v):
    wid = lax.axis_index("s") * NC + lax.axis_index("c")

    for p in range(NPASS):
        b0 = wid * B_PER_W + p * NB
        pltpu.sync_copy(x_hbm.at[pl.ds(b0 * N_IN, NB * N_IN)], x_v)

        def blk_body(j, _, b0=b0):
            c0 = j * RB
            pltpu.sync_copy(colsT_hbm.at[:, pl.ds(c0, RB)], c_v)
            pltpu.sync_copy(wT_hbm.at[:, pl.ds(c0, RB)], w_v)
            pltpu.sync_copy(bias_hbm.at[pl.ds(c0, RB)], b_v)

            def chunk_body(t, _):
                cc = t * L
                acc0 = b_v[pl.ds(cc, L)]
                accs = [acc0] * NB
                for k in range(FAN):
                    idx = c_v[k, pl.ds(cc, L)]
                    wv = w_v[k, pl.ds(cc, L)]
                    for bb in range(NB):
                        xv = plsc.load_gather(x_v, [idx + (bb * N_IN)])
                        accs[bb] = accs[bb] + xv * wv
                for bb in range(NB):
                    o_v[bb, pl.ds(cc, L)] = accs[bb]
                return 0

            lax.fori_loop(0, CHUNKS, chunk_body, 0)
            pltpu.sync_copy(o_v, out_hbm.at[pl.ds(b0, NB), pl.ds(c0, RB)])
            return 0

        lax.fori_loop(0, NBLK, blk_body, 0)


def kernel(x, rows, cols, weight, bias):
    del rows
    colsT = jnp.asarray(cols, jnp.int32).reshape(N_OUT, FAN).T
    wT = weight.reshape(N_OUT, FAN).T
    return _local_linear_sc(x.reshape(-1), colsT, wT, bias)


# speedup vs baseline: 10.1223x; 1.7198x over previous
import functools

import jax
import jax.numpy as jnp
from jax import lax
from jax.experimental import pallas as pl
from jax.experimental.pallas import tpu as pltpu
from jax.experimental.pallas import tpu_sc as plsc

B = 256
N_IN = 16384
N_OUT = 16384
FAN = 16
L = 16

NC = 2
NS = 16
NW = NC * NS
B_PER_W = B // NW
NB = 4
NPASS = B_PER_W // NB
RB = 512
NBLK = N_OUT // RB
CHUNKS = RB // L


@functools.partial(
    pl.kernel,
    out_type=jax.ShapeDtypeStruct((B, N_OUT), jnp.float32),
    mesh=plsc.VectorSubcoreMesh(core_axis_name="c", subcore_axis_name="s"),
    compiler_params=pltpu.CompilerParams(needs_layout_passes=False),
    scratch_types=[
        pltpu.VMEM((NB * N_IN,), jnp.float32),
        pltpu.VMEM((FAN, 2 * RB), jnp.int32),
        pltpu.VMEM((FAN, 2 * RB), jnp.float32),
        pltpu.VMEM((2 * RB,), jnp.float32),
        pltpu.VMEM((NB, 2 * RB), jnp.float32),
        pltpu.SemaphoreType.DMA((2,)),
        pltpu.SemaphoreType.DMA((2,)),
    ],
)
def _local_linear_sc(x_hbm, colsT_hbm, wT_hbm, bias_hbm, out_hbm,
                     x_v, c_v, w_v, b_v, o_v, sem_in, sem_out):
    wid = lax.axis_index("s") * NC + lax.axis_index("c")

    def in_copies(j, ph):
        c0 = j * RB
        return (
            pltpu.make_async_copy(
                colsT_hbm.at[:, pl.ds(c0, RB)],
                c_v.at[:, pl.ds(ph * RB, RB)], sem_in.at[ph]),
            pltpu.make_async_copy(
                wT_hbm.at[:, pl.ds(c0, RB)],
                w_v.at[:, pl.ds(ph * RB, RB)], sem_in.at[ph]),
            pltpu.make_async_copy(
                bias_hbm.at[pl.ds(c0, RB)],
                b_v.at[pl.ds(ph * RB, RB)], sem_in.at[ph]),
        )

    def out_copy(j, ph, b0):
        return pltpu.make_async_copy(
            o_v.at[:, pl.ds(ph * RB, RB)],
            out_hbm.at[pl.ds(b0, NB), pl.ds(j * RB, RB)],
            sem_out.at[ph])

    for p in range(NPASS):
        b0 = wid * B_PER_W + p * NB
        pltpu.sync_copy(x_hbm.at[pl.ds(b0 * N_IN, NB * N_IN)], x_v)
        for cp in in_copies(0, 0):
            cp.start()

        def pair_body(jp, _, b0=b0):
            for ph in range(2):
                j = jp * 2 + ph

                @pl.when(j + 1 < NBLK)
                def _prefetch(j=j, ph=ph):
                    for cp in in_copies(j + 1, 1 - ph):
                        cp.start()

                for cp in in_copies(j, ph):
                    cp.wait()

                @pl.when(j >= 2)
                def _drain(j=j, ph=ph, b0=b0):
                    out_copy(j - 2, ph, b0).wait()

                @plsc.parallel_loop(0, CHUNKS, unroll=2)
                def chunk_body(t, ph=ph):
                    cc = ph * RB + t * L
                    acc0 = b_v[pl.ds(cc, L)]
                    accs = [acc0] * NB
                    for k in range(FAN):
                        idx = c_v[k, pl.ds(cc, L)]
                        wv = w_v[k, pl.ds(cc, L)]
                        for bb in range(NB):
                            xv = plsc.load_gather(x_v, [idx + (bb * N_IN)])
                            accs[bb] = accs[bb] + xv * wv
                    for bb in range(NB):
                        o_v[bb, pl.ds(cc, L)] = accs[bb]

                out_copy(j, ph, b0).start()
            return 0

        lax.fori_loop(0, NBLK // 2, pair_body, 0)
        out_copy(NBLK - 2, 0, b0).wait()
        out_copy(NBLK - 1, 1, b0).wait()


def kernel(x, rows, cols, weight, bias):
    del rows
    colsT = jnp.asarray(cols, jnp.int32).reshape(N_OUT, FAN).T
    wT = weight.reshape(N_OUT, FAN).T
    return _local_linear_sc(x.reshape(-1), colsT, wT, bias)
